# baseline (speedup 1.0000x reference)
import functools

import jax
import jax.numpy as jnp
from jax import lax
from jax.experimental import pallas as pl
from jax.experimental.pallas import tpu as pltpu
from jax.experimental.pallas import tpu_sc as plsc

VOCAB = 50368
HIDDEN = 768
EPS = 1e-05

_NC = 2
_NS = 16
_NW = _NC * _NS
_CHUNK = 128


def _sc_gather(table, ids_flat):
    ids2, tok0, n_tok = ids_flat
    b_per_w = n_tok // _NW
    n_chunks = b_per_w // _CHUNK
    row0 = tok0 // _CHUNK
    mesh = plsc.VectorSubcoreMesh(core_axis_name="c", subcore_axis_name="s")

    @functools.partial(
        pl.kernel,
        mesh=mesh,
        out_type=jax.ShapeDtypeStruct((n_tok, HIDDEN), jnp.float32),
        scratch_types=[
            pltpu.VMEM((n_chunks, _CHUNK), jnp.int32),
            pltpu.VMEM((_CHUNK, HIDDEN), jnp.float32),
            pltpu.SemaphoreType.DMA,
        ],
    )
    def gather_kernel(table_hbm, ids_hbm, out_hbm, idx_v, rows_v, sem_g):
        wid = lax.axis_index("s") * _NC + lax.axis_index("c")
        base = wid * b_per_w
        pltpu.sync_copy(
            ids_hbm.at[pl.ds(row0 + wid * n_chunks, n_chunks)], idx_v)
        for ci in range(n_chunks):
            pltpu.async_copy(table_hbm.at[idx_v.at[ci]], rows_v, sem_g).wait()
            pltpu.sync_copy(rows_v, out_hbm.at[pl.ds(base + ci * _CHUNK, _CHUNK)])

    return gather_kernel(table, ids2)


def _ln_body(rows_ref, w_ref, out_ref):
    x = rows_ref[...]
    mean = jnp.mean(x, axis=1, keepdims=True)
    zm = x - mean
    var = jnp.mean(zm * zm, axis=1, keepdims=True)
    y = zm * lax.rsqrt(var + EPS) * w_ref[...]
    out_ref[0, :, 0, :] = y.T


def _ln_body_alias(rows_ref, w_ref, prev_ref, out_ref):
    del prev_ref
    _ln_body(rows_ref, w_ref, out_ref)


def _tc_ln_chunk(rows, w2, out_prev, bi, off, b, s):
    t = 1024
    n_tok = rows.shape[0]
    j0 = off // t
    grid = (n_tok // t,)
    in_specs = [
        pl.BlockSpec((t, HIDDEN), lambda j: (j, 0)),
        pl.BlockSpec((1, HIDDEN), lambda j: (0, 0)),
    ]
    args = [rows, w2]
    kwargs = {}
    body = _ln_body
    if out_prev is not None:
        in_specs.append(pl.BlockSpec(memory_space=pl.ANY))
        args.append(out_prev)
        kwargs["input_output_aliases"] = {2: 0}
        body = _ln_body_alias
    return pl.pallas_call(
        body,
        grid=grid,
        in_specs=in_specs,
        out_specs=pl.BlockSpec(
            (1, HIDDEN, 1, t), lambda j, bi=bi, j0=j0: (bi, 0, 0, j + j0)),
        out_shape=jax.ShapeDtypeStruct((b, HIDDEN, 1, s), jnp.float32),
        **kwargs,
    )(*args)


def kernel(input_ids, table, ln_weight):
    b, s = input_ids.shape
    w2 = ln_weight.reshape(1, HIDDEN)
    ids2 = input_ids.astype(jnp.int32).reshape(b * s // _CHUNK, _CHUNK)
    half = s // 2
    spans = [(0, 0, half), (0, half, half), (1, 0, s), (2, 0, s),
             (3, 0, half), (3, half, half)]
    out = None
    for bi, off, n_tok in spans:
        rows = _sc_gather(table, (ids2, bi * s + off, n_tok))
        out = _tc_ln_chunk(rows, w2, out, bi, off, b, s)
    return out


# speedup vs baseline: 1.0423x; 1.0423x over previous
import functools

import jax
import jax.numpy as jnp
from jax import lax
from jax.experimental import pallas as pl
from jax.experimental.pallas import tpu as pltpu
from jax.experimental.pallas import tpu_sc as plsc

VOCAB = 50368
HIDDEN = 768
EPS = 1e-05

_NC = 2
_NS = 16
_NW = _NC * _NS
_CHUNK = 128


def _sc_gather(table, ids_flat):
    ids, bi, off0, n_tok = ids_flat
    b_per_w = n_tok // _NW
    n_chunks = b_per_w // _CHUNK
    mesh = plsc.VectorSubcoreMesh(core_axis_name="c", subcore_axis_name="s")

    @functools.partial(
        pl.kernel,
        mesh=mesh,
        out_type=jax.ShapeDtypeStruct((n_tok, HIDDEN), jnp.float32),
        scratch_types=[
            pltpu.VMEM((_CHUNK,), jnp.int32),
            pltpu.VMEM((_CHUNK, HIDDEN), jnp.float32),
            pltpu.SemaphoreType.DMA,
        ],
    )
    def gather_kernel(table_hbm, ids_hbm, out_hbm, idx_v, rows_v, sem_g):
        wid = lax.axis_index("s") * _NC + lax.axis_index("c")
        base = wid * b_per_w
        for ci in range(n_chunks):
            pltpu.sync_copy(
                ids_hbm.at[bi, pl.ds(off0 + base + ci * _CHUNK, _CHUNK)], idx_v)
            pltpu.async_copy(table_hbm.at[idx_v], rows_v, sem_g).wait()
            pltpu.sync_copy(rows_v, out_hbm.at[pl.ds(base + ci * _CHUNK, _CHUNK)])

    return gather_kernel(table, ids)


def _ln_body(rows_ref, w_ref, out_ref):
    x = rows_ref[...]
    mean = jnp.mean(x, axis=1, keepdims=True)
    zm = x - mean
    var = jnp.mean(zm * zm, axis=1, keepdims=True)
    y = zm * lax.rsqrt(var + EPS) * w_ref[...]
    out_ref[0, :, 0, :] = y.T


def _ln_body_alias(rows_ref, w_ref, prev_ref, out_ref):
    del prev_ref
    _ln_body(rows_ref, w_ref, out_ref)


def _tc_ln_chunk(rows, w2, out_prev, bi, off, b, s):
    t = 2048
    n_tok = rows.shape[0]
    j0 = off // t
    grid = (n_tok // t,)
    in_specs = [
        pl.BlockSpec((t, HIDDEN), lambda j: (j, 0)),
        pl.BlockSpec((1, HIDDEN), lambda j: (0, 0)),
    ]
    args = [rows, w2]
    kwargs = {}
    body = _ln_body
    if out_prev is not None:
        in_specs.append(pl.BlockSpec(memory_space=pl.ANY))
        args.append(out_prev)
        kwargs["input_output_aliases"] = {2: 0}
        body = _ln_body_alias
    return pl.pallas_call(
        body,
        grid=grid,
        in_specs=in_specs,
        out_specs=pl.BlockSpec(
            (1, HIDDEN, 1, t), lambda j, bi=bi, j0=j0: (bi, 0, 0, j + j0)),
        out_shape=jax.ShapeDtypeStruct((b, HIDDEN, 1, s), jnp.float32),
        **kwargs,
    )(*args)


def kernel(input_ids, table, ln_weight):
    b, s = input_ids.shape
    w2 = ln_weight.reshape(1, HIDDEN)
    ids = input_ids.astype(jnp.int32)
    half = s // 2
    spans = [(0, 0, half), (0, half, half), (1, 0, s), (2, 0, s),
             (3, 0, half), (3, half, half)]
    out = None
    for bi, off, n_tok in spans:
        rows = _sc_gather(table, (ids, bi, off, n_tok))
        out = _tc_ln_chunk(rows, w2, out, bi, off, b, s)
    return out
